# baseline (speedup 1.0000x reference)
import functools

import jax
import jax.numpy as jnp
from jax import lax
from jax.experimental import pallas as pl
from jax.experimental.pallas import tpu as pltpu
from jax.experimental.pallas import tpu_sc as plsc

LANES = 16
NUM_WORKERS = 32
IDX_CHUNK = 128


def _sc_dot_kernel(batch, embed_dim):
  b_per_w = batch // NUM_WORKERS
  n_chunks = b_per_w // IDX_CHUNK
  n_dvec = embed_dim // LANES

  mesh = plsc.VectorSubcoreMesh(core_axis_name="c", subcore_axis_name="s")

  @functools.partial(
      pl.kernel,
      out_type=jax.ShapeDtypeStruct((batch,), jnp.float32),
      mesh=mesh,
      compiler_params=pltpu.CompilerParams(
          needs_layout_passes=False, use_tc_tiling_on_sc=False),
      scratch_types=[
          pltpu.VMEM((n_chunks, IDX_CHUNK), jnp.int32),
          pltpu.VMEM((n_chunks, IDX_CHUNK), jnp.int32),
          pltpu.VMEM((b_per_w, embed_dim), jnp.float32),
          pltpu.VMEM((b_per_w, embed_dim), jnp.float32),
          pltpu.VMEM((b_per_w,), jnp.float32),
          pltpu.VMEM((LANES * LANES,), jnp.float32),
          pltpu.SemaphoreType.DMA,
      ],
  )
  def kern(uidx_hbm, iidx_hbm, utab_hbm, itab_hbm, out_hbm,
           uidx_v, iidx_v, urows_v, irows_v, out_v, p_v, sem):
    wid = lax.axis_index("s") * 2 + lax.axis_index("c")
    base = wid * b_per_w

    for j in range(n_chunks):
      pltpu.sync_copy(uidx_hbm.at[pl.ds(base + j * IDX_CHUNK, IDX_CHUNK)],
                      uidx_v.at[j])
      pltpu.sync_copy(iidx_hbm.at[pl.ds(base + j * IDX_CHUNK, IDX_CHUNK)],
                      iidx_v.at[j])

    copies = []
    for j in range(n_chunks):
      copies.append(pltpu.async_copy(
          utab_hbm.at[uidx_v.at[j]],
          urows_v.at[pl.ds(j * IDX_CHUNK, IDX_CHUNK)], sem))
      copies.append(pltpu.async_copy(
          itab_hbm.at[iidx_v.at[j]],
          irows_v.at[pl.ds(j * IDX_CHUNK, IDX_CHUNK)], sem))
    for cp in copies:
      cp.wait()

    iota16 = lax.iota(jnp.int32, LANES)

    def group_body(g, _):
      rbase = g * LANES
      tot = jnp.zeros((LANES,), jnp.float32)
      for l in range(LANES):
        r = rbase + l
        acc = (urows_v[r, pl.ds(0, LANES)] * irows_v[r, pl.ds(0, LANES)])
        for d in range(1, n_dvec):
          acc = acc + (urows_v[r, pl.ds(d * LANES, LANES)] *
                       irows_v[r, pl.ds(d * LANES, LANES)])
        s = jnp.sum(acc)
        tot = jnp.where(iota16 == l, s, tot)
      out_v[pl.ds(rbase, LANES)] = tot
      return 0

    lax.fori_loop(0, b_per_w // LANES, group_body, 0)

    pltpu.sync_copy(out_v, out_hbm.at[pl.ds(base, b_per_w)])

  return kern


def kernel(user_indices, item_indices, user_weight, item_weight):
  batch = user_indices.shape[0]
  embed_dim = user_weight.shape[1]
  kern = _sc_dot_kernel(batch, embed_dim)
  return kern(user_indices.astype(jnp.int32), item_indices.astype(jnp.int32),
              user_weight, item_weight)
